# baseline (speedup 1.0000x reference)
import functools

import jax
import jax.numpy as jnp
from jax import lax
from jax.experimental import pallas as pl
from jax.experimental.pallas import tpu as pltpu
from jax.experimental.pallas import tpu_sc as plsc

R = 8000
RQ = R // 4
LK = 64
W2 = 2 * LK


def _seg_kernel(first_ref, last_ref, feats4_ref, ids4_ref,
                w1_ref, b1_ref, w2_ref, b2_ref,
                cw1_ref, cb1_ref, cw2_ref, cb2_ref, cw3_ref, cb3_ref,
                cw4_ref, cb4_ref, gum_ref, probs_ref, acc_ref, *, nb, k):
    i = pl.program_id(0)

    @pl.when(i == 0)
    def _init():
        acc_ref[...] = jnp.zeros_like(acc_ref)

    x = feats4_ref[...].astype(jnp.bfloat16)
    pf1 = jnp.maximum(jnp.dot(x, w1_ref[...], preferred_element_type=jnp.float32)
                      + b1_ref[...], 0.0).astype(jnp.bfloat16)
    pfp = jnp.maximum(jnp.dot(pf1, w2_ref[...], preferred_element_type=jnp.float32)
                      + b2_ref[...], 0.0).astype(jnp.bfloat16)

    ids4 = ids4_ref[0]
    first = first_ref[i]
    last = last_ref[i]
    w0 = first // LK
    win0 = pl.multiple_of(w0 * LK, LK)
    fast = last < win0 + W2

    @pl.when(fast)
    def _fast():
        io2 = lax.broadcasted_iota(jnp.int32, (W2, RQ), 0)
        part = jnp.zeros((W2, 64), jnp.float32)
        for t in range(4):
            oh = ((ids4[t:t + 1, :] - win0) == io2).astype(jnp.bfloat16)
            pt = jnp.dot(oh, pfp, preferred_element_type=jnp.float32)
            part = part + pt[:, 64 * t:64 * (t + 1)]
        acc_ref[pl.ds(win0, W2), :] += part

    @pl.when(jnp.logical_not(fast))
    def _slow():
        io = lax.broadcasted_iota(jnp.int32, (LK, RQ), 0)

        def body(w, carry):
            win = pl.multiple_of(w * LK, LK)
            part = jnp.zeros((LK, 64), jnp.float32)
            for t in range(4):
                oh = ((ids4[t:t + 1, :] - win) == io).astype(jnp.bfloat16)
                pt = jnp.dot(oh, pfp, preferred_element_type=jnp.float32)
                part = part + pt[:, 64 * t:64 * (t + 1)]
            acc_ref[pl.ds(win, LK), :] += part
            return carry

        lax.fori_loop(w0, last // LK + 1, body, 0)

    @pl.when(i == nb - 1)
    def _finalize():
        acc = acc_ref[pl.ds(0, k), :]
        means = acc[:, :32] / jnp.maximum(acc[:, 32:], 1.0)
        h = jnp.maximum(jnp.dot(means, cw1_ref[...],
                                preferred_element_type=jnp.float32) + cb1_ref[...], 0.0)
        h = jnp.maximum(jnp.dot(h, cw2_ref[...],
                                preferred_element_type=jnp.float32) + cb2_ref[...], 0.0)
        h = jnp.maximum(jnp.dot(h, cw3_ref[...],
                                preferred_element_type=jnp.float32) + cb3_ref[...], 0.0)
        logits = jnp.dot(h, cw4_ref[...],
                         preferred_element_type=jnp.float32) + cb4_ref[...]
        u = jnp.clip(gum_ref[...], 1e-10, 1.0 - 1e-10)
        z = logits + (-jnp.log(-jnp.log(u)))
        m = jnp.max(z, axis=1, keepdims=True)
        e = jnp.exp(z - m)
        s = jnp.sum(e, axis=1, keepdims=True)
        y0 = e[:, 0:1] / s
        y1 = e[:, 1:2] / s
        hard1 = (y1 > y0).astype(jnp.float32)
        probs_ref[...] = hard1 - y1 + y1


def _cluster_table(firsts, lasts, feats4, ids4, w1blk, b1x4, w2blk, b2x4,
                   cw1t, cb1, cw2t, cb2, cw3t, cb3, cw4t, cb4, gum):
    n4 = feats4.shape[0]
    nb = (n4 * 4) // R
    k = gum.shape[0]
    full = lambda shp: pl.BlockSpec(shp, lambda i, s=len(shp): (0,) * s)
    smem1 = pl.BlockSpec((nb,), lambda i: (0,), memory_space=pltpu.SMEM)
    return pl.pallas_call(
        functools.partial(_seg_kernel, nb=nb, k=k),
        grid=(nb,),
        in_specs=[
            smem1, smem1,
            pl.BlockSpec((RQ, 128), lambda i: (i, 0)),
            pl.BlockSpec((1, 4, RQ), lambda i: (i, 0, 0)),
            full((128, 128)), full((1, 128)), full((128, 256)), full((1, 256)),
            full((32, 32)), full((1, 32)), full((32, 32)), full((1, 32)),
            full((32, 32)), full((1, 32)), full((32, 2)), full((1, 2)),
            full((k, 2)),
        ],
        out_specs=pl.BlockSpec((k, 1), lambda i: (0, 0)),
        out_shape=jax.ShapeDtypeStruct((k, 1), jnp.float32),
        scratch_shapes=[pltpu.VMEM((k + W2, 64), jnp.float32)],
    )(firsts, lasts, feats4, ids4, w1blk, b1x4, w2blk, b2x4,
      cw1t, cb1, cw2t, cb2, cw3t, cb3, cw4t, cb4, gum)


def _gather_sc(table, ids_flat):
    n = ids_flat.shape[0]
    k = table.shape[0]
    info = plsc.get_sparse_core_info()
    nw = info.num_cores * info.num_subcores
    pt = n // nw
    mesh = plsc.VectorSubcoreMesh(core_axis_name="c", subcore_axis_name="s")

    @functools.partial(
        pl.kernel, mesh=mesh,
        compiler_params=pltpu.CompilerParams(needs_layout_passes=False),
        out_type=jax.ShapeDtypeStruct((n,), jnp.float32),
        scratch_types=[
            pltpu.VMEM((k,), jnp.float32),
            pltpu.VMEM((pt,), jnp.int32),
            pltpu.VMEM((pt,), jnp.float32),
        ],
    )
    def gk(table_hbm, ids_hbm, out_hbm, tab_v, idx_v, res_v):
        wid = lax.axis_index("s") * info.num_cores + lax.axis_index("c")
        base = wid * pt
        pltpu.sync_copy(table_hbm, tab_v)
        pltpu.sync_copy(ids_hbm.at[pl.ds(base, pt)], idx_v)

        def body(g, carry):
            idx = idx_v[pl.ds(g * 16, 16)]
            res_v[pl.ds(g * 16, 16)] = plsc.load_gather(tab_v, [idx])
            return carry

        lax.fori_loop(0, pt // 16, body, 0)
        pltpu.sync_copy(res_v, out_hbm.at[pl.ds(base, pt)])

    return gk(table, ids_flat)


def kernel(feats, cluster_ids, d_W1, d_b1, d_W2, d_b2,
           c_W1, c_b1, c_W2, c_b2, c_W3, c_b3, c_W4, c_b4, gumbel_u):
    n = feats.shape[0]
    nb = n // R
    bf16 = jnp.bfloat16
    ids_flat = cluster_ids.reshape(n)
    ids2d = ids_flat.reshape(nb, R)
    ids4 = ids_flat.reshape(nb, RQ, 4).transpose(0, 2, 1)
    feats4 = feats.reshape(n // 4, 128)
    eye4 = jnp.eye(4, dtype=jnp.float32)
    w2ext = jnp.concatenate([d_W2.T, jnp.zeros((32, 32), jnp.float32)], axis=1)
    b2ext = jnp.concatenate([d_b2, jnp.ones((32,), jnp.float32)])
    w1blk = jnp.kron(eye4, d_W1.T).astype(bf16)
    w2blk = jnp.kron(eye4, w2ext).astype(bf16)
    b1x4 = jnp.tile(d_b1, 4).reshape(1, 128)
    b2x4 = jnp.tile(b2ext, 4).reshape(1, 256)
    probs = _cluster_table(
        ids2d[:, 0], ids2d[:, R - 1], feats4, ids4,
        w1blk, b1x4, w2blk, b2x4,
        c_W1.T, c_b1.reshape(1, 32), c_W2.T, c_b2.reshape(1, 32),
        c_W3.T, c_b3.reshape(1, 32), c_W4.T, c_b4.reshape(1, 2),
        gumbel_u)
    out = _gather_sc(probs.reshape(gumbel_u.shape[0]), ids_flat)
    return out.reshape(n, 1)


# speedup vs baseline: 1.2199x; 1.2199x over previous
import functools

import jax
import jax.numpy as jnp
from jax import lax
from jax.experimental import pallas as pl
from jax.experimental.pallas import tpu as pltpu
from jax.experimental.pallas import tpu_sc as plsc

R = 8000
LK = 64
W2 = 2 * LK


def _seg_kernel(first_ref, last_ref, feats_ref, ids_ref,
                dw1_ref, db1_ref, dw2_ref, db2_ref,
                cw1_ref, cb1_ref, cw2_ref, cb2_ref, cw3_ref, cb3_ref,
                cw4_ref, cb4_ref, gum_ref, probs_ref, acc_ref, *, nb, k):
    i = pl.program_id(0)

    @pl.when(i == 0)
    def _init():
        acc_ref[...] = jnp.zeros_like(acc_ref)

    def mlp(rows):
        xb = rows.astype(jnp.bfloat16)
        pf = jnp.maximum(jnp.dot(xb, dw1_ref[...], preferred_element_type=jnp.float32)
                         + db1_ref[...], 0.0).astype(jnp.bfloat16)
        return jnp.maximum(jnp.dot(pf, dw2_ref[...], preferred_element_type=jnp.float32)
                           + db2_ref[...], 0.0).astype(jnp.bfloat16)

    ids_row = ids_ref[0]
    first = first_ref[i]
    last = last_ref[i]
    w0 = first // LK
    win0 = pl.multiple_of(w0 * LK, LK)
    fast = last < win0 + W2

    @pl.when(fast)
    def _fast():
        pf_ext = mlp(feats_ref[...])
        io2 = lax.broadcasted_iota(jnp.int32, (W2, R), 0)
        oh = ((ids_row - win0) == io2).astype(jnp.bfloat16)
        part = jnp.dot(oh, pf_ext, preferred_element_type=jnp.float32)
        acc_ref[pl.ds(win0, W2), :] += part

    @pl.when(jnp.logical_not(fast))
    def _slow():
        pf_ext = mlp(feats_ref[...])
        io = lax.broadcasted_iota(jnp.int32, (LK, R), 0)

        def body(w, carry):
            win = pl.multiple_of(w * LK, LK)
            oh = ((ids_row - win) == io).astype(jnp.bfloat16)
            p = jnp.dot(oh, pf_ext, preferred_element_type=jnp.float32)
            acc_ref[pl.ds(win, LK), :] += p
            return carry

        lax.fori_loop(w0, last // LK + 1, body, 0)

    @pl.when(i == nb - 1)
    def _finalize():
        acc = acc_ref[pl.ds(0, k), :]
        means = acc[:, :32] / jnp.maximum(acc[:, 32:], 1.0)
        h = jnp.maximum(jnp.dot(means, cw1_ref[...],
                                preferred_element_type=jnp.float32) + cb1_ref[...], 0.0)
        h = jnp.maximum(jnp.dot(h, cw2_ref[...],
                                preferred_element_type=jnp.float32) + cb2_ref[...], 0.0)
        h = jnp.maximum(jnp.dot(h, cw3_ref[...],
                                preferred_element_type=jnp.float32) + cb3_ref[...], 0.0)
        logits = jnp.dot(h, cw4_ref[...],
                         preferred_element_type=jnp.float32) + cb4_ref[...]
        u = jnp.clip(gum_ref[...], 1e-10, 1.0 - 1e-10)
        z = logits + (-jnp.log(-jnp.log(u)))
        m = jnp.max(z, axis=1, keepdims=True)
        e = jnp.exp(z - m)
        s = jnp.sum(e, axis=1, keepdims=True)
        y0 = e[:, 0:1] / s
        y1 = e[:, 1:2] / s
        hard1 = (y1 > y0).astype(jnp.float32)
        probs_ref[...] = hard1 - y1 + y1


def _cluster_table(firsts, lasts, feats, ids3d, dw1t, db1, dw2t, db2,
                   cw1t, cb1, cw2t, cb2, cw3t, cb3, cw4t, cb4, gum):
    n = feats.shape[0]
    nb = n // R
    k = gum.shape[0]
    full = lambda shp: pl.BlockSpec(shp, lambda i, s=len(shp): (0,) * s)
    smem1 = pl.BlockSpec((nb,), lambda i: (0,), memory_space=pltpu.SMEM)
    return pl.pallas_call(
        functools.partial(_seg_kernel, nb=nb, k=k),
        grid=(nb,),
        in_specs=[
            smem1, smem1,
            pl.BlockSpec((R, 32), lambda i: (i, 0)),
            pl.BlockSpec((1, 1, R), lambda i: (i, 0, 0)),
            full((32, 32)), full((1, 32)), full((32, 64)), full((1, 64)),
            full((32, 32)), full((1, 32)), full((32, 32)), full((1, 32)),
            full((32, 32)), full((1, 32)), full((32, 2)), full((1, 2)),
            full((k, 2)),
        ],
        out_specs=pl.BlockSpec((k, 1), lambda i: (0, 0)),
        out_shape=jax.ShapeDtypeStruct((k, 1), jnp.float32),
        scratch_shapes=[pltpu.VMEM((k + W2, 64), jnp.float32)],
    )(firsts, lasts, feats, ids3d, dw1t, db1, dw2t, db2,
      cw1t, cb1, cw2t, cb2, cw3t, cb3, cw4t, cb4, gum)


def _gather_sc(table, ids_flat):
    n = ids_flat.shape[0]
    k = table.shape[0]
    info = plsc.get_sparse_core_info()
    nw = info.num_cores * info.num_subcores
    pt = n // nw
    mesh = plsc.VectorSubcoreMesh(core_axis_name="c", subcore_axis_name="s")

    @functools.partial(
        pl.kernel, mesh=mesh,
        compiler_params=pltpu.CompilerParams(needs_layout_passes=False),
        out_type=jax.ShapeDtypeStruct((n,), jnp.float32),
        scratch_types=[
            pltpu.VMEM((k,), jnp.float32),
            pltpu.VMEM((pt,), jnp.int32),
            pltpu.VMEM((pt,), jnp.float32),
        ],
    )
    def gk(table_hbm, ids_hbm, out_hbm, tab_v, idx_v, res_v):
        wid = lax.axis_index("s") * info.num_cores + lax.axis_index("c")
        base = wid * pt
        pltpu.sync_copy(table_hbm, tab_v)
        pltpu.sync_copy(ids_hbm.at[pl.ds(base, pt)], idx_v)

        def body(g, carry):
            idx = idx_v[pl.ds(g * 16, 16)]
            res_v[pl.ds(g * 16, 16)] = plsc.load_gather(tab_v, [idx])
            return carry

        lax.fori_loop(0, pt // 16, body, 0)
        pltpu.sync_copy(res_v, out_hbm.at[pl.ds(base, pt)])

    return gk(table, ids_flat)


def kernel(feats, cluster_ids, d_W1, d_b1, d_W2, d_b2,
           c_W1, c_b1, c_W2, c_b2, c_W3, c_b3, c_W4, c_b4, gumbel_u):
    n = feats.shape[0]
    nb = n // R
    bf16 = jnp.bfloat16
    ids_flat = cluster_ids.reshape(n)
    ids3d = ids_flat.reshape(nb, 1, R)
    ids2d = ids_flat.reshape(nb, R)
    w2ext = jnp.concatenate([d_W2.T, jnp.zeros((32, 32), jnp.float32)], axis=1)
    b2ext = jnp.concatenate([d_b2, jnp.ones((32,), jnp.float32)]).reshape(1, 64)
    probs = _cluster_table(
        ids2d[:, 0], ids2d[:, R - 1], feats, ids3d,
        d_W1.T.astype(bf16), d_b1.reshape(1, 32),
        w2ext.astype(bf16), b2ext,
        c_W1.T, c_b1.reshape(1, 32), c_W2.T, c_b2.reshape(1, 32),
        c_W3.T, c_b3.reshape(1, 32), c_W4.T, c_b4.reshape(1, 2),
        gumbel_u)
    out = _gather_sc(probs.reshape(gumbel_u.shape[0]), ids_flat)
    return out.reshape(n, 1)
